# baseline (speedup 1.0000x reference)
import functools

import jax
import jax.numpy as jnp
from jax import lax
from jax.experimental import pallas as pl
from jax.experimental.pallas import tpu as pltpu
from jax.experimental.pallas import tpu_sc as plsc

_N = 100000
_F = 150000
_NC = 2
_NS = 16
_NW = _NC * _NS
_PER_W = 3200
_NPAD = _NW * _PER_W
_B = 800
_NCH = _PER_W // _B
_GRP = _B // 16
_D = 8


def _sc_body(table_h, cf_h, unv_h, rho_h, rhs_h, cont_h, out_h,
             idx_v, unv_v, rho_v, rhs_v, ra_v, rb_v, cont_v, out_v,
             sem_s, sem_a, sem_b):
    wid = lax.axis_index("s") * _NC + lax.axis_index("c")
    wbase = wid * _PER_W

    stg = [
        pltpu.async_copy(cf_h.at[:, pl.ds(wbase, _PER_W)], idx_v, sem_s),
        pltpu.async_copy(unv_h.at[pl.ds(wbase, _PER_W), :], unv_v, sem_s),
        pltpu.async_copy(rho_h.at[pl.ds(wbase, _PER_W)], rho_v, sem_s),
        pltpu.async_copy(rhs_h.at[pl.ds(wbase, _PER_W)], rhs_v, sem_s),
    ]
    for h in stg:
        h.wait()

    bufs = (ra_v, rb_v)
    sems = (sem_a, sem_b)

    def fire(ch):
        buf, sem = bufs[ch % 2], sems[ch % 2]
        return [
            pltpu.async_copy(
                table_h.at[idx_v.at[k, pl.ds(ch * _B, _B)]],
                buf.at[pl.ds(k * _B, _B), :], sem)
            for k in range(3)
        ]

    def compute(ch, r_v):
        def group(g, carry):
            off = ch * _B + g * 16
            cells = off + lax.iota(jnp.int32, 16)
            lanes = g * 16 + lax.iota(jnp.int32, 16)

            def col(rows, j):
                return plsc.load_gather(
                    r_v, [rows, jnp.full((16,), j, jnp.int32)])

            cont = jnp.zeros((16,), jnp.float32)
            fa0 = jnp.zeros((16,), jnp.float32)
            fa1 = jnp.zeros((16,), jnp.float32)
            fp0 = jnp.zeros((16,), jnp.float32)
            fp1 = jnp.zeros((16,), jnp.float32)
            fd0 = jnp.zeros((16,), jnp.float32)
            fd1 = jnp.zeros((16,), jnp.float32)
            for k in range(3):
                rows = k * _B + lanes
                u0 = col(rows, 0)
                u1 = col(rows, 1)
                p = col(rows, 2)
                g0 = col(rows, 3)
                g1 = col(rows, 4)
                ar = col(rows, 5)
                nx = plsc.load_gather(
                    unv_v, [cells, jnp.full((16,), 2 * k, jnp.int32)])
                ny = plsc.load_gather(
                    unv_v, [cells, jnp.full((16,), 2 * k + 1, jnp.int32)])
                da = (u0 * nx + u1 * ny) * ar
                pa = p * ar
                cont = cont + da
                fa0 = fa0 + u0 * da
                fa1 = fa1 + u1 * da
                fp0 = fp0 + pa * nx
                fp1 = fp1 + pa * ny
                fd0 = fd0 + g0
                fd1 = fd1 + g1
            inv = 1.0 / rho_v[pl.ds(off, 16)]
            rc = rhs_v[pl.ds(off, 16)]
            o0 = rc * (-fa0 - fp0 * inv) + fd0
            o1 = rc * (-fa1 - fp1 * inv) + fd1
            cont_v[pl.ds(off, 16)] = cont
            plsc.store_scatter(
                out_v, [cells, jnp.zeros((16,), jnp.int32)], o0)
            plsc.store_scatter(
                out_v, [cells, jnp.ones((16,), jnp.int32)], o1)
            return carry

        lax.fori_loop(0, _GRP, group, 0)

    hs = [fire(0), fire(1)]
    for ch in range(_NCH):
        for h in hs[ch]:
            h.wait()
        compute(ch, bufs[ch % 2])
        if ch + 2 < _NCH:
            hs.append(fire(ch + 2))

    pltpu.sync_copy(cont_v, cont_h.at[pl.ds(wbase, _PER_W)])
    pltpu.sync_copy(out_v, out_h.at[pl.ds(wbase, _PER_W)])


_sc_call = functools.partial(
    pl.kernel,
    mesh=plsc.VectorSubcoreMesh(core_axis_name="c", subcore_axis_name="s"),
    compiler_params=pltpu.CompilerParams(
        needs_layout_passes=False, use_tc_tiling_on_sc=False),
    out_type=[
        jax.ShapeDtypeStruct((_NPAD,), jnp.float32),
        jax.ShapeDtypeStruct((_NPAD, 2), jnp.float32),
    ],
    scratch_types=[
        pltpu.VMEM((3, _PER_W), jnp.int32),
        pltpu.VMEM((_PER_W, 6), jnp.float32),
        pltpu.VMEM((_PER_W,), jnp.float32),
        pltpu.VMEM((_PER_W,), jnp.float32),
        pltpu.VMEM((3 * _B, _D), jnp.float32),
        pltpu.VMEM((3 * _B, _D), jnp.float32),
        pltpu.VMEM((_PER_W,), jnp.float32),
        pltpu.VMEM((_PER_W, 2), jnp.float32),
        pltpu.SemaphoreType.DMA,
        pltpu.SemaphoreType.DMA,
        pltpu.SemaphoreType.DMA,
    ],
)(_sc_body)


def kernel(uv_face, p_face, flux_D, unv, rho, rhs_coef, face_area, cell_face):
    table = jnp.concatenate(
        [uv_face, p_face, flux_D, face_area,
         jnp.zeros((_F, 2), jnp.float32)], axis=1)
    pad = _NPAD - _N
    cf = jnp.pad(cell_face, ((0, 0), (0, pad)))
    unv_p = jnp.pad(unv.reshape(_N, 6), ((0, pad), (0, 0)))
    rho_p = jnp.pad(rho.reshape(_N), (0, pad), constant_values=1.0)
    rhs_p = jnp.pad(rhs_coef.reshape(_N), (0, pad))
    cont, out = _sc_call(table, cf, unv_p, rho_p, rhs_p)
    return cont[:_N].reshape(_N, 1), out[:_N]


# speedup vs baseline: 1.3108x; 1.3108x over previous
import functools

import jax
import jax.numpy as jnp
from jax import lax
from jax.experimental import pallas as pl
from jax.experimental.pallas import tpu as pltpu
from jax.experimental.pallas import tpu_sc as plsc

_N = 100000
_F = 150000
_NC = 2
_NS = 16
_NW = _NC * _NS
_PER_W = 3200
_NPAD = _NW * _PER_W
_B = 800
_NCH = _PER_W // _B
_GRP = _B // 16
_D = 8


def _sc_body(table_h, idx_h, ops_h, cont_h, out_h,
             idx_v, ops_v, ra_v, rb_v, cont_v, out_v,
             sem_s, sem_a, sem_b):
    wid = lax.axis_index("s") * _NC + lax.axis_index("c")
    wbase = wid * _PER_W

    s0 = pltpu.async_copy(
        idx_h.at[pl.ds(wid * _NCH * 3 * _B, _NCH * 3 * _B)], idx_v, sem_s)
    s1 = pltpu.async_copy(
        ops_h.at[:, pl.ds(wbase, _PER_W)], ops_v, sem_s)
    s0.wait()
    s1.wait()

    bufs = (ra_v, rb_v)
    sems = (sem_a, sem_b)

    def fire(ch):
        return pltpu.async_copy(
            table_h.at[idx_v.at[pl.ds(ch * 3 * _B, 3 * _B)]],
            bufs[ch % 2], sems[ch % 2])

    def compute(ch, r_v):
        def group(g, carry):
            off = ch * _B + g * 16
            cells = g * 16 + lax.iota(jnp.int32, 16)

            def col(rows, j):
                return plsc.load_gather(
                    r_v, [rows, jnp.full((16,), j, jnp.int32)])

            cont = jnp.zeros((16,), jnp.float32)
            fa0 = jnp.zeros((16,), jnp.float32)
            fa1 = jnp.zeros((16,), jnp.float32)
            fp0 = jnp.zeros((16,), jnp.float32)
            fp1 = jnp.zeros((16,), jnp.float32)
            fd0 = jnp.zeros((16,), jnp.float32)
            fd1 = jnp.zeros((16,), jnp.float32)
            for k in range(3):
                rows = k * _B + cells
                u0 = col(rows, 0)
                u1 = col(rows, 1)
                p = col(rows, 2)
                g0 = col(rows, 3)
                g1 = col(rows, 4)
                ar = col(rows, 5)
                nx = ops_v[2 * k, pl.ds(off, 16)]
                ny = ops_v[2 * k + 1, pl.ds(off, 16)]
                da = (u0 * nx + u1 * ny) * ar
                pa = p * ar
                cont = cont + da
                fa0 = fa0 + u0 * da
                fa1 = fa1 + u1 * da
                fp0 = fp0 + pa * nx
                fp1 = fp1 + pa * ny
                fd0 = fd0 + g0
                fd1 = fd1 + g1
            inv = 1.0 / ops_v[6, pl.ds(off, 16)]
            rc = ops_v[7, pl.ds(off, 16)]
            o0 = rc * (-fa0 - fp0 * inv) + fd0
            o1 = rc * (-fa1 - fp1 * inv) + fd1
            cont_v[pl.ds(off, 16)] = cont
            ocells = ch * _B + cells
            plsc.store_scatter(
                out_v, [ocells, jnp.zeros((16,), jnp.int32)], o0)
            plsc.store_scatter(
                out_v, [ocells, jnp.ones((16,), jnp.int32)], o1)
            return carry

        lax.fori_loop(0, _GRP, group, 0)

    hs = [fire(0), fire(1)]
    for ch in range(_NCH):
        hs[ch].wait()
        compute(ch, bufs[ch % 2])
        if ch + 2 < _NCH:
            hs.append(fire(ch + 2))

    pltpu.sync_copy(cont_v, cont_h.at[pl.ds(wbase, _PER_W)])
    pltpu.sync_copy(out_v, out_h.at[pl.ds(wbase, _PER_W)])


_sc_call = functools.partial(
    pl.kernel,
    mesh=plsc.VectorSubcoreMesh(core_axis_name="c", subcore_axis_name="s"),
    compiler_params=pltpu.CompilerParams(
        needs_layout_passes=False, use_tc_tiling_on_sc=False),
    out_type=[
        jax.ShapeDtypeStruct((_NPAD,), jnp.float32),
        jax.ShapeDtypeStruct((_NPAD, 2), jnp.float32),
    ],
    scratch_types=[
        pltpu.VMEM((_NCH * 3 * _B,), jnp.int32),
        pltpu.VMEM((8, _PER_W), jnp.float32),
        pltpu.VMEM((3 * _B, _D), jnp.float32),
        pltpu.VMEM((3 * _B, _D), jnp.float32),
        pltpu.VMEM((_PER_W,), jnp.float32),
        pltpu.VMEM((_PER_W, 2), jnp.float32),
        pltpu.SemaphoreType.DMA,
        pltpu.SemaphoreType.DMA,
        pltpu.SemaphoreType.DMA,
    ],
)(_sc_body)


def kernel(uv_face, p_face, flux_D, unv, rho, rhs_coef, face_area, cell_face):
    table = jnp.concatenate(
        [uv_face, p_face, flux_D, face_area,
         jnp.zeros((_F, 2), jnp.float32)], axis=1)
    pad = _NPAD - _N
    cf = jnp.pad(cell_face, ((0, 0), (0, pad)))
    idx = cf.reshape(3, _NW, _NCH, _B).transpose(1, 2, 0, 3).reshape(-1)
    rho_p = jnp.pad(rho.reshape(1, _N), ((0, 0), (0, pad)),
                    constant_values=1.0)
    ops = jnp.concatenate(
        [jnp.pad(unv.reshape(_N, 6).T, ((0, 0), (0, pad))),
         rho_p,
         jnp.pad(rhs_coef.reshape(1, _N), ((0, 0), (0, pad)))], axis=0)
    cont, out = _sc_call(table, idx, ops)
    return cont[:_N].reshape(_N, 1), out[:_N]


# speedup vs baseline: 1.3115x; 1.0006x over previous
import functools

import jax
import jax.numpy as jnp
from jax import lax
from jax.experimental import pallas as pl
from jax.experimental.pallas import tpu as pltpu
from jax.experimental.pallas import tpu_sc as plsc

_N = 100000
_F = 150000
_NC = 2
_NS = 16
_NW = _NC * _NS
_PER_W = 3200
_NPAD = _NW * _PER_W
_B = 400
_NCH = _PER_W // _B
_GRP = _B // 16
_D = 8


def _sc_body(table_h, idx_h, ops_h, cont_h, out_h,
             idx_v, ops_v, ra_v, rb_v, rc_v, rd_v, cont_v, out_v,
             sem_s, sem_a, sem_b, sem_c, sem_d):
    wid = lax.axis_index("s") * _NC + lax.axis_index("c")
    wbase = wid * _PER_W

    s0 = pltpu.async_copy(
        idx_h.at[pl.ds(wid * _NCH * 3 * _B, _NCH * 3 * _B)], idx_v, sem_s)
    s1 = pltpu.async_copy(
        ops_h.at[:, pl.ds(wbase, _PER_W)], ops_v, sem_s)
    s0.wait()
    s1.wait()

    bufs = (ra_v, rb_v, rc_v, rd_v)
    sems = (sem_a, sem_b, sem_c, sem_d)
    nbuf = len(bufs)

    def fire(ch):
        return pltpu.async_copy(
            table_h.at[idx_v.at[pl.ds(ch * 3 * _B, 3 * _B)]],
            bufs[ch % nbuf], sems[ch % nbuf])

    def compute(ch, r_v):
        def group(g, carry):
            off = ch * _B + g * 16
            cells = g * 16 + lax.iota(jnp.int32, 16)

            def col(rows, j):
                return plsc.load_gather(
                    r_v, [rows, jnp.full((16,), j, jnp.int32)])

            cont = jnp.zeros((16,), jnp.float32)
            fa0 = jnp.zeros((16,), jnp.float32)
            fa1 = jnp.zeros((16,), jnp.float32)
            fp0 = jnp.zeros((16,), jnp.float32)
            fp1 = jnp.zeros((16,), jnp.float32)
            fd0 = jnp.zeros((16,), jnp.float32)
            fd1 = jnp.zeros((16,), jnp.float32)
            for k in range(3):
                rows = k * _B + cells
                u0 = col(rows, 0)
                u1 = col(rows, 1)
                p = col(rows, 2)
                g0 = col(rows, 3)
                g1 = col(rows, 4)
                ar = col(rows, 5)
                nx = ops_v[2 * k, pl.ds(off, 16)]
                ny = ops_v[2 * k + 1, pl.ds(off, 16)]
                da = (u0 * nx + u1 * ny) * ar
                pa = p * ar
                cont = cont + da
                fa0 = fa0 + u0 * da
                fa1 = fa1 + u1 * da
                fp0 = fp0 + pa * nx
                fp1 = fp1 + pa * ny
                fd0 = fd0 + g0
                fd1 = fd1 + g1
            inv = 1.0 / ops_v[6, pl.ds(off, 16)]
            rc = ops_v[7, pl.ds(off, 16)]
            o0 = rc * (-fa0 - fp0 * inv) + fd0
            o1 = rc * (-fa1 - fp1 * inv) + fd1
            cont_v[pl.ds(off, 16)] = cont
            ocells = ch * _B + cells
            plsc.store_scatter(
                out_v, [ocells, jnp.zeros((16,), jnp.int32)], o0)
            plsc.store_scatter(
                out_v, [ocells, jnp.ones((16,), jnp.int32)], o1)
            return carry

        lax.fori_loop(0, _GRP, group, 0)

    hs = [fire(ch) for ch in range(nbuf)]
    for ch in range(_NCH):
        hs[ch].wait()
        compute(ch, bufs[ch % nbuf])
        if ch + nbuf < _NCH:
            hs.append(fire(ch + nbuf))

    pltpu.sync_copy(cont_v, cont_h.at[pl.ds(wbase, _PER_W)])
    pltpu.sync_copy(out_v, out_h.at[pl.ds(wbase, _PER_W)])


_sc_call = functools.partial(
    pl.kernel,
    mesh=plsc.VectorSubcoreMesh(core_axis_name="c", subcore_axis_name="s"),
    compiler_params=pltpu.CompilerParams(
        needs_layout_passes=False, use_tc_tiling_on_sc=False),
    out_type=[
        jax.ShapeDtypeStruct((_NPAD,), jnp.float32),
        jax.ShapeDtypeStruct((_NPAD, 2), jnp.float32),
    ],
    scratch_types=[
        pltpu.VMEM((_NCH * 3 * _B,), jnp.int32),
        pltpu.VMEM((8, _PER_W), jnp.float32),
        pltpu.VMEM((3 * _B, _D), jnp.float32),
        pltpu.VMEM((3 * _B, _D), jnp.float32),
        pltpu.VMEM((3 * _B, _D), jnp.float32),
        pltpu.VMEM((3 * _B, _D), jnp.float32),
        pltpu.VMEM((_PER_W,), jnp.float32),
        pltpu.VMEM((_PER_W, 2), jnp.float32),
        pltpu.SemaphoreType.DMA,
        pltpu.SemaphoreType.DMA,
        pltpu.SemaphoreType.DMA,
        pltpu.SemaphoreType.DMA,
        pltpu.SemaphoreType.DMA,
    ],
)(_sc_body)


def kernel(uv_face, p_face, flux_D, unv, rho, rhs_coef, face_area, cell_face):
    table = jnp.concatenate(
        [uv_face, p_face, flux_D, face_area,
         jnp.zeros((_F, 2), jnp.float32)], axis=1)
    pad = _NPAD - _N
    cf = jnp.pad(cell_face, ((0, 0), (0, pad)))
    idx = cf.reshape(3, _NW, _NCH, _B).transpose(1, 2, 0, 3).reshape(-1)
    rho_p = jnp.pad(rho.reshape(1, _N), ((0, 0), (0, pad)),
                    constant_values=1.0)
    ops = jnp.concatenate(
        [jnp.pad(unv.reshape(_N, 6).T, ((0, 0), (0, pad))),
         rho_p,
         jnp.pad(rhs_coef.reshape(1, _N), ((0, 0), (0, pad)))], axis=0)
    cont, out = _sc_call(table, idx, ops)
    return cont[:_N].reshape(_N, 1), out[:_N]
